# baseline (speedup 1.0000x reference)
import functools

import jax
import jax.numpy as jnp
from jax import lax
from jax.experimental import pallas as pl
from jax.experimental.pallas import tpu as pltpu
from jax.experimental.pallas import tpu_sc as plsc

_EPS = 1e-06
_NC = 2
_NS = 16


def _proj_body(scores_ref, values_ref, mask_ref, w_ref, b_ref, out_ref):
    s = scores_ref[...]
    v = values_ref[...]
    weighted = jnp.sum(s[:, :, None] * v, axis=1)
    proj = jnp.dot(weighted, w_ref[...],
                   preferred_element_type=jnp.float32)
    out_ref[...] = (proj + b_ref[...]) * mask_ref[...]


def _project(scores, values, mask, W, b):
    M, K, R = values.shape
    D = W.shape[1]
    BM = 512
    return pl.pallas_call(
        _proj_body,
        grid=(M // BM,),
        in_specs=[
            pl.BlockSpec((BM, K), lambda i: (i, 0)),
            pl.BlockSpec((BM, K, R), lambda i: (i, 0, 0)),
            pl.BlockSpec((BM, 1), lambda i: (i, 0)),
            pl.BlockSpec((R, D), lambda i: (0, 0)),
            pl.BlockSpec((1, D), lambda i: (0, 0)),
        ],
        out_specs=pl.BlockSpec((BM, D), lambda i: (i, 0)),
        out_shape=jax.ShapeDtypeStruct((M, D), jnp.float32),
    )(scores, values, mask.reshape(M, 1), W, b.reshape(1, D))


def _ln_body(x_ref, g_ref, b_ref, o_ref):
    x = x_ref[...]
    mean = jnp.mean(x, axis=-1, keepdims=True)
    xc = x - mean
    var = jnp.mean(xc * xc, axis=-1, keepdims=True)
    o_ref[...] = xc * lax.rsqrt(var + _EPS) * g_ref[...] + b_ref[...]


def _layernorm(x, g, b, block_rows):
    N, D = x.shape
    return pl.pallas_call(
        _ln_body,
        grid=(N // block_rows,),
        in_specs=[
            pl.BlockSpec((block_rows, D), lambda i: (i, 0)),
            pl.BlockSpec((1, D), lambda i: (0, 0)),
            pl.BlockSpec((1, D), lambda i: (0, 0)),
        ],
        out_specs=pl.BlockSpec((block_rows, D), lambda i: (i, 0)),
        out_shape=jax.ShapeDtypeStruct((N, D), jnp.float32),
    )(x, g.reshape(1, D), b.reshape(1, D))


def _sc_aggregate(enc2d, proj, bp, sp, T):
    BT = enc2d.shape[0]
    M, D = proj.shape
    CCH = 128
    cols_per_core = D // _NC
    n_chunks = cols_per_core // CCH
    m_per_tile = M // _NS
    r_per_tile = BT // _NS

    mesh = plsc.VectorSubcoreMesh(core_axis_name="c", subcore_axis_name="s")

    @functools.partial(
        pl.kernel,
        out_type=jax.ShapeDtypeStruct((M, D), jnp.float32),
        mesh=mesh,
        scratch_types=[
            pltpu.VMEM_SHARED((BT, CCH), jnp.float32),
            pltpu.VMEM((m_per_tile, CCH), jnp.float32),
            pltpu.VMEM((m_per_tile, CCH), jnp.float32),
            pltpu.VMEM((m_per_tile,), jnp.int32),
            pltpu.VMEM((m_per_tile,), jnp.int32),
            pltpu.SemaphoreType.DMA,
            pltpu.SemaphoreType.DMA,
        ],
    )
    def k(enc_hbm, proj_hbm, bp_hbm, sp_hbm, comp_hbm,
          slab, ev, pv, idx_v, sp_v, sem0, sem1):
        c = lax.axis_index("c")
        s = lax.axis_index("s")
        m0 = pl.multiple_of(s * m_per_tile, m_per_tile)
        r0 = pl.multiple_of(s * r_per_tile, r_per_tile)
        pltpu.sync_copy(bp_hbm.at[pl.ds(m0, m_per_tile)], idx_v)
        pltpu.sync_copy(sp_hbm.at[pl.ds(m0, m_per_tile)], sp_v)
        for j in range(m_per_tile // 16):
            sl = pl.ds(j * 16, 16)
            idx_v[sl] = idx_v[sl] * T + sp_v[sl]
        for ch in range(n_chunks):
            c0 = pl.multiple_of(c * cols_per_core + ch * CCH, CCH)
            with jax.named_scope("seed"):
                half = r_per_tile // 2
                r1 = pl.multiple_of(r0 + half, half)
                pend0 = pltpu.async_copy(
                    enc_hbm.at[pl.ds(r0, half), pl.ds(c0, CCH)],
                    slab.at[pl.ds(r0, half)],
                    sem0,
                )
                pend1 = pltpu.async_copy(
                    enc_hbm.at[pl.ds(r1, half), pl.ds(c0, CCH)],
                    slab.at[pl.ds(r1, half)],
                    sem1,
                )
                pltpu.sync_copy(
                    proj_hbm.at[pl.ds(m0, m_per_tile), pl.ds(c0, CCH)], pv)
                pend0.wait()
                pend1.wait()
                plsc.subcore_barrier()
            with jax.named_scope("scatter_add"):
                pltpu.sync_copy(pv, slab.at[idx_v], add=True)
                plsc.subcore_barrier()
            with jax.named_scope("gather_back"):
                pltpu.sync_copy(slab.at[idx_v], ev)
                pltpu.sync_copy(
                    ev, comp_hbm.at[pl.ds(m0, m_per_tile), pl.ds(c0, CCH)])
                plsc.subcore_barrier()

    return k(enc2d, proj, bp, sp)


def _sc_patch(ln_compact, bp, sp, T, out_ref):
    M, D = ln_compact.shape
    m_per_w = M // (_NC * _NS)

    mesh = plsc.VectorSubcoreMesh(core_axis_name="c", subcore_axis_name="s")

    @functools.partial(
        pl.kernel,
        out_type=(),
        mesh=mesh,
        scratch_types=[
            pltpu.VMEM((m_per_w, D), jnp.float32),
            pltpu.VMEM((m_per_w,), jnp.int32),
            pltpu.VMEM((m_per_w,), jnp.int32),
        ],
    )
    def k(lnc_hbm, bp_hbm, sp_hbm, out_hbm, xv, idx_v, sp_v):
        c = lax.axis_index("c")
        s = lax.axis_index("s")
        w = s * _NC + c
        m0 = pl.multiple_of(w * m_per_w, m_per_w)
        pltpu.sync_copy(bp_hbm.at[pl.ds(m0, m_per_w)], idx_v)
        pltpu.sync_copy(sp_hbm.at[pl.ds(m0, m_per_w)], sp_v)
        for j in range(m_per_w // 16):
            sl = pl.ds(j * 16, 16)
            idx_v[sl] = idx_v[sl] * T + sp_v[sl]
        pltpu.sync_copy(lnc_hbm.at[pl.ds(m0, m_per_w)], xv)
        pltpu.sync_copy(xv, out_hbm.at[idx_v])

    k(ln_compact, bp, sp, out_ref)


def kernel(encoded_input, retrieval_values, retrieval_scores,
           mention_batch_positions, mention_start_positions,
           mention_end_positions, mention_mask, deterministic,
           W, b, ln_scale, ln_bias):
    B, T, D = encoded_input.shape
    BT = B * T
    bp = mention_batch_positions.astype(jnp.int32)
    sp = mention_start_positions.astype(jnp.int32)
    proj = _project(retrieval_scores, retrieval_values, mention_mask, W, b)
    enc2d = encoded_input.reshape(BT, D)
    ln_base = _layernorm(enc2d, ln_scale, ln_bias, 512)
    compact = _sc_aggregate(enc2d, proj, bp, sp, T)
    ln_compact = _layernorm(compact, ln_scale, ln_bias, 512)
    out_ref = jax.new_ref(ln_base)
    _sc_patch(ln_compact, bp, sp, T, out_ref)
    return out_ref[...].reshape(B, T, D)


# speedup vs baseline: 1.0008x; 1.0008x over previous
import functools

import jax
import jax.numpy as jnp
from jax import lax
from jax.experimental import pallas as pl
from jax.experimental.pallas import tpu as pltpu
from jax.experimental.pallas import tpu_sc as plsc

_EPS = 1e-06
_NC = 2
_NS = 16


def _proj_body(scores_ref, values_ref, w_ref, out_ref):
    s = scores_ref[...]
    v = values_ref[...]
    K = v.shape[1]
    weighted = jnp.sum(s[:, :K, None] * v, axis=1)
    waug = jnp.concatenate([weighted, s[:, K:]], axis=1)
    out_ref[...] = jnp.dot(waug, w_ref[...],
                           preferred_element_type=jnp.float32)


def _project(scores, values, mask, W, b):
    M, K, R = values.shape
    D = W.shape[1]
    BM = 256
    scores_aug = jnp.concatenate(
        [scores * mask[:, None], mask[:, None]], axis=1)
    w_aug = jnp.concatenate([W, b[None, :]], axis=0)
    return pl.pallas_call(
        _proj_body,
        grid=(M // BM,),
        in_specs=[
            pl.BlockSpec((BM, K + 1), lambda i: (i, 0)),
            pl.BlockSpec((BM, K, R), lambda i: (i, 0, 0)),
            pl.BlockSpec((R + 1, D), lambda i: (0, 0)),
        ],
        out_specs=pl.BlockSpec((BM, D), lambda i: (i, 0)),
        out_shape=jax.ShapeDtypeStruct((M, D), jnp.float32),
    )(scores_aug, values, w_aug)


def _ln_body(x_ref, g_ref, b_ref, o_ref):
    x = x_ref[...]
    mean = jnp.mean(x, axis=-1, keepdims=True)
    xc = x - mean
    var = jnp.mean(xc * xc, axis=-1, keepdims=True)
    o_ref[...] = xc * lax.rsqrt(var + _EPS) * g_ref[...] + b_ref[...]


def _layernorm(x, g, b, block_rows):
    N, D = x.shape
    return pl.pallas_call(
        _ln_body,
        grid=(N // block_rows,),
        in_specs=[
            pl.BlockSpec((block_rows, D), lambda i: (i, 0)),
            pl.BlockSpec((1, D), lambda i: (0, 0)),
            pl.BlockSpec((1, D), lambda i: (0, 0)),
        ],
        out_specs=pl.BlockSpec((block_rows, D), lambda i: (i, 0)),
        out_shape=jax.ShapeDtypeStruct((N, D), jnp.float32),
    )(x, g.reshape(1, D), b.reshape(1, D))


def _sc_aggregate(enc2d, proj, bp, sp, T):
    BT = enc2d.shape[0]
    M, D = proj.shape
    CCH = 128
    cols_per_core = D // _NC
    n_chunks = cols_per_core // CCH
    m_per_tile = M // _NS
    r_per_tile = BT // _NS

    mesh = plsc.VectorSubcoreMesh(core_axis_name="c", subcore_axis_name="s")

    @functools.partial(
        pl.kernel,
        out_type=jax.ShapeDtypeStruct((M, D), jnp.float32),
        mesh=mesh,
        scratch_types=[
            pltpu.VMEM_SHARED((BT, CCH), jnp.float32),
            pltpu.VMEM((m_per_tile, CCH), jnp.float32),
            pltpu.VMEM((m_per_tile, CCH), jnp.float32),
            pltpu.VMEM((m_per_tile,), jnp.int32),
            pltpu.VMEM((m_per_tile,), jnp.int32),
            pltpu.SemaphoreType.DMA,
            pltpu.SemaphoreType.DMA,
        ],
    )
    def k(enc_hbm, proj_hbm, bp_hbm, sp_hbm, comp_hbm,
          slab, ev, pv, idx_v, sp_v, sem0, sem1):
        c = lax.axis_index("c")
        s = lax.axis_index("s")
        m0 = pl.multiple_of(s * m_per_tile, m_per_tile)
        r0 = pl.multiple_of(s * r_per_tile, r_per_tile)
        pltpu.sync_copy(bp_hbm.at[pl.ds(m0, m_per_tile)], idx_v)
        pltpu.sync_copy(sp_hbm.at[pl.ds(m0, m_per_tile)], sp_v)
        for j in range(m_per_tile // 16):
            sl = pl.ds(j * 16, 16)
            idx_v[sl] = idx_v[sl] * T + sp_v[sl]
        for ch in range(n_chunks):
            c0 = pl.multiple_of(c * cols_per_core + ch * CCH, CCH)
            with jax.named_scope("seed"):
                pending = pltpu.async_copy(
                    enc_hbm.at[pl.ds(r0, r_per_tile), pl.ds(c0, CCH)],
                    slab.at[pl.ds(r0, r_per_tile)],
                    sem0,
                )
                pltpu.sync_copy(
                    proj_hbm.at[pl.ds(m0, m_per_tile), pl.ds(c0, CCH)], pv)
                pending.wait()
                plsc.subcore_barrier()
            with jax.named_scope("scatter_add"):
                pltpu.sync_copy(pv, slab.at[idx_v], add=True)
                plsc.subcore_barrier()
            with jax.named_scope("gather_back"):
                pltpu.sync_copy(slab.at[idx_v], ev)
                pltpu.sync_copy(
                    ev, comp_hbm.at[pl.ds(m0, m_per_tile), pl.ds(c0, CCH)])
                plsc.subcore_barrier()

    return k(enc2d, proj, bp, sp)


def _sc_patch(ln_compact, bp, sp, T, out_ref):
    M, D = ln_compact.shape
    m_per_w = M // (_NC * _NS)

    mesh = plsc.VectorSubcoreMesh(core_axis_name="c", subcore_axis_name="s")

    @functools.partial(
        pl.kernel,
        out_type=(),
        mesh=mesh,
        scratch_types=[
            pltpu.VMEM((m_per_w, D), jnp.float32),
            pltpu.VMEM((m_per_w,), jnp.int32),
            pltpu.VMEM((m_per_w,), jnp.int32),
        ],
    )
    def k(lnc_hbm, bp_hbm, sp_hbm, out_hbm, xv, idx_v, sp_v):
        c = lax.axis_index("c")
        s = lax.axis_index("s")
        w = s * _NC + c
        m0 = pl.multiple_of(w * m_per_w, m_per_w)
        pltpu.sync_copy(bp_hbm.at[pl.ds(m0, m_per_w)], idx_v)
        pltpu.sync_copy(sp_hbm.at[pl.ds(m0, m_per_w)], sp_v)
        for j in range(m_per_w // 16):
            sl = pl.ds(j * 16, 16)
            idx_v[sl] = idx_v[sl] * T + sp_v[sl]
        pltpu.sync_copy(lnc_hbm.at[pl.ds(m0, m_per_w)], xv)
        pltpu.sync_copy(xv, out_hbm.at[idx_v])

    k(ln_compact, bp, sp, out_ref)


def kernel(encoded_input, retrieval_values, retrieval_scores,
           mention_batch_positions, mention_start_positions,
           mention_end_positions, mention_mask, deterministic,
           W, b, ln_scale, ln_bias):
    B, T, D = encoded_input.shape
    BT = B * T
    bp = mention_batch_positions.astype(jnp.int32)
    sp = mention_start_positions.astype(jnp.int32)
    proj = _project(retrieval_scores, retrieval_values, mention_mask, W, b)
    enc2d = encoded_input.reshape(BT, D)
    ln_base = _layernorm(enc2d, ln_scale, ln_bias, 512)
    compact = _sc_aggregate(enc2d, proj, bp, sp, T)
    ln_compact = _layernorm(compact, ln_scale, ln_bias, 512)
    out_ref = jax.new_ref(ln_base)
    _sc_patch(ln_compact, bp, sp, T, out_ref)
    return out_ref[...].reshape(B, T, D)


# speedup vs baseline: 1.0189x; 1.0180x over previous
import functools

import jax
import jax.numpy as jnp
from jax import lax
from jax.experimental import pallas as pl
from jax.experimental.pallas import tpu as pltpu
from jax.experimental.pallas import tpu_sc as plsc

_EPS = 1e-06
_NC = 2
_NS = 16


def _proj_body(scores_ref, values_ref, mask_ref, w_ref, b_ref, out_ref):
    s = scores_ref[...]
    v = values_ref[...]
    weighted = jnp.sum(s[:, :, None] * v, axis=1)
    proj = jnp.dot(weighted, w_ref[...],
                   preferred_element_type=jnp.float32)
    out_ref[...] = (proj + b_ref[...]) * mask_ref[...]


def _project(scores, values, mask, W, b):
    M, K, R = values.shape
    D = W.shape[1]
    BM = 256
    return pl.pallas_call(
        _proj_body,
        grid=(M // BM,),
        in_specs=[
            pl.BlockSpec((BM, K), lambda i: (i, 0)),
            pl.BlockSpec((BM, K, R), lambda i: (i, 0, 0)),
            pl.BlockSpec((BM, 1), lambda i: (i, 0)),
            pl.BlockSpec((R, D), lambda i: (0, 0)),
            pl.BlockSpec((1, D), lambda i: (0, 0)),
        ],
        out_specs=pl.BlockSpec((BM, D), lambda i: (i, 0)),
        out_shape=jax.ShapeDtypeStruct((M, D), jnp.float32),
    )(scores, values, mask.reshape(M, 1), W, b.reshape(1, D))


def _ln_body(x_ref, g_ref, b_ref, o_ref):
    x = x_ref[...]
    mean = jnp.mean(x, axis=-1, keepdims=True)
    xc = x - mean
    var = jnp.mean(xc * xc, axis=-1, keepdims=True)
    o_ref[...] = xc * lax.rsqrt(var + _EPS) * g_ref[...] + b_ref[...]


def _layernorm(x, g, b, block_rows):
    N, D = x.shape
    return pl.pallas_call(
        _ln_body,
        grid=(N // block_rows,),
        in_specs=[
            pl.BlockSpec((block_rows, D), lambda i: (i, 0)),
            pl.BlockSpec((1, D), lambda i: (0, 0)),
            pl.BlockSpec((1, D), lambda i: (0, 0)),
        ],
        out_specs=pl.BlockSpec((block_rows, D), lambda i: (i, 0)),
        out_shape=jax.ShapeDtypeStruct((N, D), jnp.float32),
    )(x, g.reshape(1, D), b.reshape(1, D))


def _sc_aggregate(enc2d, proj, rows):
    BT = enc2d.shape[0]
    M, D = proj.shape
    CCH = 128
    cols_per_core = D // _NC
    n_chunks = cols_per_core // CCH
    m_per_tile = M // _NS
    r_per_tile = BT // _NS

    mesh = plsc.VectorSubcoreMesh(core_axis_name="c", subcore_axis_name="s")

    @functools.partial(
        pl.kernel,
        out_type=jax.ShapeDtypeStruct((M, D), jnp.float32),
        mesh=mesh,
        scratch_types=[
            pltpu.VMEM_SHARED((BT, CCH), jnp.float32),
            pltpu.VMEM((m_per_tile, CCH), jnp.float32),
            pltpu.VMEM((m_per_tile, CCH), jnp.float32),
            pltpu.VMEM((m_per_tile,), jnp.int32),
            pltpu.SemaphoreType.DMA,
        ],
    )
    def k(enc_hbm, proj_hbm, rows_hbm, comp_hbm, slab, ev, pv, idx_v, sem0):
        c = lax.axis_index("c")
        s = lax.axis_index("s")
        m0 = pl.multiple_of(s * m_per_tile, m_per_tile)
        r0 = pl.multiple_of(s * r_per_tile, r_per_tile)
        pltpu.sync_copy(rows_hbm.at[pl.ds(m0, m_per_tile)], idx_v)
        for ch in range(n_chunks):
            c0 = pl.multiple_of(c * cols_per_core + ch * CCH, CCH)
            with jax.named_scope("seed"):
                pending = pltpu.async_copy(
                    enc_hbm.at[pl.ds(r0, r_per_tile), pl.ds(c0, CCH)],
                    slab.at[pl.ds(r0, r_per_tile)],
                    sem0,
                )
                pltpu.sync_copy(
                    proj_hbm.at[pl.ds(m0, m_per_tile), pl.ds(c0, CCH)], pv)
                pending.wait()
                plsc.subcore_barrier()
            with jax.named_scope("scatter_add"):
                pltpu.sync_copy(pv, slab.at[idx_v], add=True)
                plsc.subcore_barrier()
            with jax.named_scope("gather_back"):
                pltpu.sync_copy(slab.at[idx_v], ev)
                pltpu.sync_copy(
                    ev, comp_hbm.at[pl.ds(m0, m_per_tile), pl.ds(c0, CCH)])
                plsc.subcore_barrier()

    return k(enc2d, proj, rows)


def _sc_patch(ln_compact, rows, out_ref):
    M, D = ln_compact.shape
    m_per_w = M // (_NC * _NS)

    mesh = plsc.VectorSubcoreMesh(core_axis_name="c", subcore_axis_name="s")

    @functools.partial(
        pl.kernel,
        out_type=(),
        mesh=mesh,
        scratch_types=[
            pltpu.VMEM((m_per_w, D), jnp.float32),
            pltpu.VMEM((m_per_w,), jnp.int32),
        ],
    )
    def k(lnc_hbm, rows_hbm, out_hbm, xv, idx_v):
        c = lax.axis_index("c")
        s = lax.axis_index("s")
        w = s * _NC + c
        m0 = pl.multiple_of(w * m_per_w, m_per_w)
        pltpu.sync_copy(rows_hbm.at[pl.ds(m0, m_per_w)], idx_v)
        pltpu.sync_copy(lnc_hbm.at[pl.ds(m0, m_per_w)], xv)
        pltpu.sync_copy(xv, out_hbm.at[idx_v])

    k(ln_compact, rows, out_ref)


def kernel(encoded_input, retrieval_values, retrieval_scores,
           mention_batch_positions, mention_start_positions,
           mention_end_positions, mention_mask, deterministic,
           W, b, ln_scale, ln_bias):
    B, T, D = encoded_input.shape
    BT = B * T
    rows = (mention_batch_positions.astype(jnp.int32) * T
            + mention_start_positions.astype(jnp.int32))
    proj = _project(retrieval_scores, retrieval_values, mention_mask, W, b)
    enc2d = encoded_input.reshape(BT, D)
    ln_base = _layernorm(enc2d, ln_scale, ln_bias, 512)
    compact = _sc_aggregate(enc2d, proj, rows)
    ln_compact = _layernorm(compact, ln_scale, ln_bias, 512)
    out_ref = jax.new_ref(ln_base)
    _sc_patch(ln_compact, rows, out_ref)
    return out_ref[...].reshape(B, T, D)


# speedup vs baseline: 1.0224x; 1.0035x over previous
import functools

import jax
import jax.numpy as jnp
from jax import lax
from jax.experimental import pallas as pl
from jax.experimental.pallas import tpu as pltpu
from jax.experimental.pallas import tpu_sc as plsc

_EPS = 1e-06
_NC = 2
_NS = 16


def _proj_body(scores_ref, values_ref, mask_ref, w_ref, b_ref, out_ref):
    s = scores_ref[...]
    v = values_ref[...]
    weighted = jnp.sum(s[:, :, None] * v, axis=1)
    proj = jnp.dot(weighted, w_ref[...],
                   preferred_element_type=jnp.float32)
    out_ref[...] = (proj + b_ref[...]) * mask_ref[...]


def _project(scores, values, mask, W, b):
    M, K, R = values.shape
    D = W.shape[1]
    BM = 256
    return pl.pallas_call(
        _proj_body,
        grid=(M // BM,),
        in_specs=[
            pl.BlockSpec((BM, K), lambda i: (i, 0)),
            pl.BlockSpec((BM, K, R), lambda i: (i, 0, 0)),
            pl.BlockSpec((BM, 1), lambda i: (i, 0)),
            pl.BlockSpec((R, D), lambda i: (0, 0)),
            pl.BlockSpec((1, D), lambda i: (0, 0)),
        ],
        out_specs=pl.BlockSpec((BM, D), lambda i: (i, 0)),
        out_shape=jax.ShapeDtypeStruct((M, D), jnp.float32),
    )(scores, values, mask.reshape(M, 1), W, b.reshape(1, D))


def _ln_body(x_ref, g_ref, b_ref, o_ref):
    x = x_ref[...]
    mean = jnp.mean(x, axis=-1, keepdims=True)
    xc = x - mean
    var = jnp.mean(xc * xc, axis=-1, keepdims=True)
    o_ref[...] = xc * lax.rsqrt(var + _EPS) * g_ref[...] + b_ref[...]


def _layernorm(x, g, b, block_rows):
    N, D = x.shape
    return pl.pallas_call(
        _ln_body,
        grid=(N // block_rows,),
        in_specs=[
            pl.BlockSpec((block_rows, D), lambda i: (i, 0)),
            pl.BlockSpec((1, D), lambda i: (0, 0)),
            pl.BlockSpec((1, D), lambda i: (0, 0)),
        ],
        out_specs=pl.BlockSpec((block_rows, D), lambda i: (i, 0)),
        out_shape=jax.ShapeDtypeStruct((N, D), jnp.float32),
    )(x, g.reshape(1, D), b.reshape(1, D))


def _sc_aggregate(enc2d, proj, rows):
    BT = enc2d.shape[0]
    M, D = proj.shape
    CCH = 128
    cols_per_core = D // _NC
    n_chunks = cols_per_core // CCH
    m_per_tile = M // _NS
    r_per_tile = BT // _NS

    mesh = plsc.VectorSubcoreMesh(core_axis_name="c", subcore_axis_name="s")

    @functools.partial(
        pl.kernel,
        out_type=jax.ShapeDtypeStruct((M, D), jnp.float32),
        mesh=mesh,
        scratch_types=[
            pltpu.VMEM_SHARED((BT, CCH), jnp.float32),
            pltpu.VMEM((m_per_tile, CCH), jnp.float32),
            pltpu.VMEM((m_per_tile, CCH), jnp.float32),
            pltpu.VMEM((m_per_tile,), jnp.int32),
            pltpu.SemaphoreType.DMA,
        ],
    )
    def k(enc_hbm, proj_hbm, rows_hbm, comp_hbm, slab, ev, pv, idx_v, sem0):
        c = lax.axis_index("c")
        s = lax.axis_index("s")
        m0 = pl.multiple_of(s * m_per_tile, m_per_tile)
        r0 = pl.multiple_of(s * r_per_tile, r_per_tile)
        pltpu.sync_copy(rows_hbm.at[pl.ds(m0, m_per_tile)], idx_v)
        for ch in range(n_chunks):
            c0 = pl.multiple_of(c * cols_per_core + ch * CCH, CCH)
            pending = pltpu.async_copy(
                enc_hbm.at[pl.ds(r0, r_per_tile), pl.ds(c0, CCH)],
                slab.at[pl.ds(r0, r_per_tile)],
                sem0,
            )
            pltpu.sync_copy(
                proj_hbm.at[pl.ds(m0, m_per_tile), pl.ds(c0, CCH)], pv)
            pending.wait()
            plsc.subcore_barrier()
            pltpu.sync_copy(pv, slab.at[idx_v], add=True)
            plsc.subcore_barrier()
            pltpu.sync_copy(slab.at[idx_v], ev)
            pltpu.sync_copy(
                ev, comp_hbm.at[pl.ds(m0, m_per_tile), pl.ds(c0, CCH)])
            plsc.subcore_barrier()

    return k(enc2d, proj, rows)


def _sc_patch(ln_compact, rows, out_ref):
    M, D = ln_compact.shape
    m_per_w = M // (_NC * _NS)

    mesh = plsc.VectorSubcoreMesh(core_axis_name="c", subcore_axis_name="s")

    @functools.partial(
        pl.kernel,
        out_type=(),
        mesh=mesh,
        scratch_types=[
            pltpu.VMEM((m_per_w, D), jnp.float32),
            pltpu.VMEM((m_per_w,), jnp.int32),
        ],
    )
    def k(lnc_hbm, rows_hbm, out_hbm, xv, idx_v):
        c = lax.axis_index("c")
        s = lax.axis_index("s")
        w = s * _NC + c
        m0 = pl.multiple_of(w * m_per_w, m_per_w)
        pltpu.sync_copy(rows_hbm.at[pl.ds(m0, m_per_w)], idx_v)
        pltpu.sync_copy(lnc_hbm.at[pl.ds(m0, m_per_w)], xv)
        pltpu.sync_copy(xv, out_hbm.at[idx_v])

    k(ln_compact, rows, out_ref)


def kernel(encoded_input, retrieval_values, retrieval_scores,
           mention_batch_positions, mention_start_positions,
           mention_end_positions, mention_mask, deterministic,
           W, b, ln_scale, ln_bias):
    B, T, D = encoded_input.shape
    BT = B * T
    rows = (mention_batch_positions.astype(jnp.int32) * T
            + mention_start_positions.astype(jnp.int32))
    proj = _project(retrieval_scores, retrieval_values, mention_mask, W, b)
    enc2d = encoded_input.reshape(BT, D)
    ln_base = _layernorm(enc2d, ln_scale, ln_bias, 512)
    compact = _sc_aggregate(enc2d, proj, rows)
    ln_compact = _layernorm(compact, ln_scale, ln_bias, 512)
    out_ref = jax.new_ref(ln_base)
    _sc_patch(ln_compact, rows, out_ref)
    return out_ref[...].reshape(B, T, D)
